# Initial kernel scaffold; baseline (speedup 1.0000x reference)
import jax
import jax.numpy as jnp
from jax.experimental import pallas as pl


def kernel(indices, table):
    raise NotImplementedError("write your pallas kernel here")


# speedup vs baseline: 5.9563x; 5.9563x over previous
import functools

import jax
import jax.numpy as jnp
from jax import lax
from jax.experimental import pallas as pl
from jax.experimental.pallas import tpu as pltpu
from jax.experimental.pallas import tpu_sc as plsc

B = 4096
L = 50
EMBED_DIM = 128
NUM_CORES = 2
NUM_SUBCORES = 16
NUM_WORKERS = NUM_CORES * NUM_SUBCORES
B_PER_W = B // NUM_WORKERS
CB = 8
CHUNK_I = CB * L
N_CHUNKS = B_PER_W // CB
NV = EMBED_DIM // 16


def _sc_body(idx_hbm, table_hbm, out_hbm, idx_v, rows_v, out_v, sem):
    wid = lax.axis_index("s") * NUM_CORES + lax.axis_index("c")
    idx_base = wid * (B_PER_W * L)
    out_base = wid * B_PER_W

    def chunk_body(c, carry):
        pltpu.sync_copy(idx_hbm.at[pl.ds(idx_base + c * CHUNK_I, CHUNK_I)],
                        idx_v)
        pltpu.async_copy(table_hbm.at[idx_v], rows_v, sem).wait()

        def example_body(e, carry2):
            row0 = e * L
            accs = [jnp.zeros((16,), jnp.float32) for _ in range(NV)]
            for l in range(L):
                for d in range(NV):
                    accs[d] = accs[d] + rows_v[row0 + l, pl.ds(d * 16, 16)]
            for d in range(NV):
                out_v[e, pl.ds(d * 16, 16)] = accs[d]
            return carry2

        lax.fori_loop(0, CB, example_body, 0)
        pltpu.sync_copy(out_v, out_hbm.at[pl.ds(out_base + c * CB, CB)])
        return carry

    lax.fori_loop(0, N_CHUNKS, chunk_body, 0)


@jax.jit
def kernel(indices, table):
    idx_flat = indices.reshape(-1).astype(jnp.int32)
    run = pl.kernel(
        _sc_body,
        out_type=jax.ShapeDtypeStruct((B, EMBED_DIM), jnp.float32),
        mesh=plsc.VectorSubcoreMesh(core_axis_name="c", subcore_axis_name="s"),
        scratch_types=[
            pltpu.VMEM((CHUNK_I,), jnp.int32),
            pltpu.VMEM((CHUNK_I, EMBED_DIM), jnp.float32),
            pltpu.VMEM((CB, EMBED_DIM), jnp.float32),
            pltpu.SemaphoreType.DMA,
        ],
    )
    return run(idx_flat, table)


# speedup vs baseline: 8.5695x; 1.4387x over previous
import functools

import jax
import jax.numpy as jnp
from jax import lax
from jax.experimental import pallas as pl
from jax.experimental.pallas import tpu as pltpu
from jax.experimental.pallas import tpu_sc as plsc

B = 4096
L = 50
EMBED_DIM = 128
NUM_CORES = 2
NUM_SUBCORES = 16
NUM_WORKERS = NUM_CORES * NUM_SUBCORES
B_PER_W = B // NUM_WORKERS
CB = 4
CHUNK_I = CB * L
N_CHUNKS = B_PER_W // CB
NV = EMBED_DIM // 16


def _sc_body(idx_hbm, table_hbm, out_hbm, idx_all, rows0, rows1, out_v,
             sem0, sem1):
    wid = lax.axis_index("s") * NUM_CORES + lax.axis_index("c")
    idx_base = wid * (B_PER_W * L)
    out_base = wid * B_PER_W

    pltpu.sync_copy(idx_hbm.at[pl.ds(idx_base, B_PER_W * L)], idx_all)

    bufs = ((rows0, sem0), (rows1, sem1))

    def issue(c, buf, sem):
        pltpu.async_copy(
            table_hbm.at[idx_all.at[pl.ds(c * CHUNK_I, CHUNK_I)]], buf, sem)

    issue(0, rows0, sem0)
    issue(1, rows1, sem1)

    def accumulate(rows_v, c):
        def example_body(e, carry):
            row0 = e * L
            accs = [jnp.zeros((16,), jnp.float32) for _ in range(NV)]
            for l in range(L):
                for d in range(NV):
                    accs[d] = accs[d] + rows_v[row0 + l, pl.ds(d * 16, 16)]
            for d in range(NV):
                out_v[c * CB + e, pl.ds(d * 16, 16)] = accs[d]
            return carry

        lax.fori_loop(0, CB, example_body, 0)

    def pair_body(cc, carry):
        for b in range(2):
            c = cc * 2 + b
            rows_v, sem = bufs[b]
            pltpu.make_async_copy(
                table_hbm.at[idx_all.at[pl.ds(0, CHUNK_I)]], rows_v,
                sem).wait()
            accumulate(rows_v, c)

            @pl.when(c + 2 < N_CHUNKS)
            def _():
                issue(c + 2, rows_v, sem)
        return carry

    lax.fori_loop(0, N_CHUNKS // 2, pair_body, 0)
    pltpu.sync_copy(out_v, out_hbm.at[pl.ds(out_base, B_PER_W)])


@jax.jit
def kernel(indices, table):
    idx_flat = indices.reshape(-1).astype(jnp.int32)
    run = pl.kernel(
        _sc_body,
        out_type=jax.ShapeDtypeStruct((B, EMBED_DIM), jnp.float32),
        mesh=plsc.VectorSubcoreMesh(core_axis_name="c", subcore_axis_name="s"),
        scratch_types=[
            pltpu.VMEM((B_PER_W * L,), jnp.int32),
            pltpu.VMEM((CHUNK_I, EMBED_DIM), jnp.float32),
            pltpu.VMEM((CHUNK_I, EMBED_DIM), jnp.float32),
            pltpu.VMEM((B_PER_W, EMBED_DIM), jnp.float32),
            pltpu.SemaphoreType.DMA,
            pltpu.SemaphoreType.DMA,
        ],
    )
    return run(idx_flat, table)
